# Initial kernel scaffold; baseline (speedup 1.0000x reference)
import jax
import jax.numpy as jnp
from jax.experimental import pallas as pl


def kernel(x, w1_0, b1_0, w2_0, b2_0, wd_0, bd_0, w1_1, b1_1, w2_1, b2_1, wd_1, bd_1, w1_2, b1_2, w2_2, b2_2, wd_2, bd_2):
    raise NotImplementedError("write your pallas kernel here")


# speedup vs baseline: 1.6455x; 1.6455x over previous
import functools

import jax
import jax.numpy as jnp
from jax import lax
from jax.experimental import pallas as pl

K = 2


def _tcn_body(T, BS, x_ref,
              wm0, bm0, w2c0, b2_0,
              wm1, bm1, w2c1, b2_1,
              wm2, bm2, w2c2, b2_2,
              out_ref):
    M = BS * T
    X = jnp.concatenate([x_ref[j] for j in range(BS)], axis=-1)
    tmod = lax.broadcasted_iota(jnp.int32, (1, M), 1) % T

    def shift(h, d):
        c = h.shape[0]
        sh = jnp.concatenate([jnp.zeros((c, d), jnp.float32), h[:, :-d]], axis=1)
        return jnp.where(tmod >= d, sh, 0.0)

    def block(h, wm, bm, w2c, b2, d, co):
        x2 = jnp.concatenate([shift(h, d), h], axis=0)
        y = jnp.dot(wm[...], x2, preferred_element_type=jnp.float32) + bm[...]
        h1 = jax.nn.relu(y[:co])
        res = y[co:]
        x2b = jnp.concatenate([shift(h1, d), h1], axis=0)
        o2 = jax.nn.relu(
            jnp.dot(w2c[...], x2b, preferred_element_type=jnp.float32) + b2[...])
        return jax.nn.relu(o2 + res)

    h = block(X, wm0[...], bm0[...], w2c0, b2_0, 1, 32)
    h = block(h, wm1[...], bm1[...], w2c1, b2_1, 2, 16)
    h = block(h, wm2[...], bm2[...], w2c2, b2_2, 4, 64)

    for j in range(BS):
        out_ref[j] = h[:, j * T:(j + 1) * T]


def _prep_layer(w1, b1, w2, b2, wd, bd):
    co, ci, _ = w1.shape
    top = jnp.concatenate([w1[:, :, 0], w1[:, :, 1]], axis=1)
    bot = jnp.concatenate([jnp.zeros((co, ci), w1.dtype), wd[:, :, 0]], axis=1)
    wm = jnp.concatenate([top, bot], axis=0)
    bm = jnp.concatenate([b1, bd])[:, None]
    w2c = jnp.concatenate([w2[:, :, 0], w2[:, :, 1]], axis=1)
    return wm, bm, w2c, b2[:, None]


def kernel(x, w1_0, b1_0, w2_0, b2_0, wd_0, bd_0,
           w1_1, b1_1, w2_1, b2_1, wd_1, bd_1,
           w1_2, b1_2, w2_2, b2_2, wd_2, bd_2):
    B, CIN, T = x.shape
    BS = 8
    CO = w1_2.shape[0]

    wargs = (_prep_layer(w1_0, b1_0, w2_0, b2_0, wd_0, bd_0)
             + _prep_layer(w1_1, b1_1, w2_1, b2_1, wd_1, bd_1)
             + _prep_layer(w1_2, b1_2, w2_2, b2_2, wd_2, bd_2))

    grid = B // BS
    body = functools.partial(_tcn_body, T, BS)
    out = pl.pallas_call(
        body,
        grid=(grid,),
        in_specs=[pl.BlockSpec((BS, CIN, T), lambda i: (i, 0, 0))]
                 + [pl.BlockSpec(w.shape, lambda i: tuple(0 for _ in w.shape))
                    for w in wargs],
        out_specs=pl.BlockSpec((BS, CO, T), lambda i: (i, 0, 0)),
        out_shape=jax.ShapeDtypeStruct((B, CO, T), jnp.float32),
    )(x, *wargs)
    return out
